# baseline (speedup 1.0000x reference)
import functools

import jax
import jax.numpy as jnp
from jax import lax
from jax.experimental import pallas as pl
from jax.experimental.pallas import tpu as pltpu
from jax.experimental.pallas import tpu_sc as plsc

N = 8192
D = 64
K = 10
OUT = 128


_BR = 256


def _topk_body(pcdT_ref, pcd_blk_ref, idx_ref):
    pcdT = pcdT_ref[...]
    a = pcd_blk_ref[...]
    sq_col = jnp.sum(pcdT * pcdT, axis=0, keepdims=True)
    sq_row = jnp.sum(a * a, axis=1, keepdims=True)
    dot = jnp.dot(a, pcdT, preferred_element_type=jnp.float32)
    dd = (sq_row + sq_col) - 2.0 * dot

    iota_row = lax.broadcasted_iota(jnp.int32, (1, N), 1)
    big = jnp.int32(N)
    cols = []
    for _ in range(K):
        m = jnp.min(dd, axis=1, keepdims=True)
        cand = jnp.where(dd == m, iota_row, big)
        j = jnp.min(cand, axis=1, keepdims=True)
        cols.append(j)
        dd = jnp.where(iota_row == j, jnp.inf, dd)
    idx_ref[...] = jnp.concatenate(cols, axis=1)


def _topk(pcdT, pcd):
    grid = (N // _BR,)
    return pl.pallas_call(
        _topk_body,
        grid=grid,
        in_specs=[
            pl.BlockSpec((D, N), lambda i: (0, 0)),
            pl.BlockSpec((_BR, D), lambda i: (i, 0)),
        ],
        out_specs=pl.BlockSpec((_BR, K), lambda i: (i, 0)),
        out_shape=jax.ShapeDtypeStruct((N, K), jnp.int32),
    )(pcdT, pcd)


_DEPTH = 4
_G = N // 128


def _topk_fast_body(pcdT_ref, pcd_blk_ref, out_ref):
    pcdT = pcdT_ref[...]
    a = pcd_blk_ref[...]
    sq_col = jnp.sum(pcdT * pcdT, axis=0, keepdims=True)
    dot = jnp.dot(a, pcdT, preferred_element_type=jnp.float32)
    dd = sq_col - 2.0 * dot

    lane = lax.broadcasted_iota(jnp.int32, (1, 128), 1)
    inf = jnp.float32(jnp.inf)
    big = jnp.int32(1 << 30)

    sv = [jnp.full((_BR, 128), inf, jnp.float32) for _ in range(_DEPTH)]
    sa = [jnp.zeros((_BR, 128), jnp.int32) for _ in range(_DEPTH)]
    rej = jnp.full((_BR, 128), inf, jnp.float32)
    for g in range(_G):
        v = dd[:, g * 128:(g + 1) * 128]
        vid = jnp.full((1, 1), g, jnp.int32)
        for t in range(_DEPTH):
            c = v < sv[t]
            sv[t], v = jnp.where(c, v, sv[t]), jnp.where(c, sv[t], v)
            sa[t], vid = jnp.where(c, vid, sa[t]), jnp.where(c, sa[t], vid)
        rej = jnp.minimum(rej, v)

    gid = [sa[t] * 128 + lane for t in range(_DEPTH)]
    cv = jnp.concatenate(sv, axis=1)
    ci = jnp.concatenate(gid, axis=1)
    ct = cv.T
    it = ci.T

    rows = []
    m = None
    for _ in range(K):
        m = jnp.min(ct, axis=0, keepdims=True)
        cand = jnp.where(ct == m, it, big)
        j = jnp.min(cand, axis=0, keepdims=True)
        rows.append(j)
        ct = jnp.where(it == j, inf, ct)

    rejt = rej.T
    rejmin = jnp.min(rejt, axis=0, keepdims=True)
    rowflag = (rejmin <= m).astype(jnp.int32)
    for _ in range(16 - K):
        rows.append(rowflag)
    out_ref[...] = jnp.concatenate(rows, axis=0).T


def _topk_fast(pcdT, pcd):
    grid = (N // _BR,)
    return pl.pallas_call(
        _topk_fast_body,
        grid=grid,
        in_specs=[
            pl.BlockSpec((D, N), lambda i: (0, 0)),
            pl.BlockSpec((_BR, D), lambda i: (i, 0)),
        ],
        out_specs=pl.BlockSpec((_BR, 16), lambda i: (i, 0)),
        out_shape=jax.ShapeDtypeStruct((N, 16), jnp.int32),
    )(pcdT, pcd)


_B = N * K
_CHUNK = 1280


def _make_sc_gather():
    info = plsc.get_sparse_core_info()
    nc, ns = info.num_cores, info.num_subcores
    nw = nc * ns
    b_per_w = _B // nw
    n_chunks = b_per_w // _CHUNK
    mesh = plsc.VectorSubcoreMesh(core_axis_name="c", subcore_axis_name="s")

    @functools.partial(
        pl.kernel,
        mesh=mesh,
        compiler_params=pltpu.CompilerParams(use_tc_tiling_on_sc=False),
        out_type=jax.ShapeDtypeStruct((_B, D), jnp.float32),
        scratch_types=[
            pltpu.VMEM((_CHUNK,), jnp.int32),
            pltpu.VMEM((_CHUNK, D), jnp.float32),
            pltpu.SemaphoreType.DMA,
        ],
    )
    def gather(table_hbm, idx_hbm, out_hbm, idx_v, rows_v, sem):
        wid = lax.axis_index("s") * nc + lax.axis_index("c")
        base = wid * b_per_w
        for c in range(n_chunks):
            off = base + c * _CHUNK
            pltpu.sync_copy(idx_hbm.at[pl.ds(off, _CHUNK)], idx_v)
            pltpu.async_copy(table_hbm.at[idx_v], rows_v, sem).wait()
            pltpu.sync_copy(rows_v, out_hbm.at[pl.ds(off, _CHUNK)])

    return gather


_BR3 = 512


def _conv_body(v_ref, w2_ref, b_ref, out_ref):
    v = v_ref[...]
    w2 = w2_ref[...]
    acc = jnp.dot(v, w2, preferred_element_type=jnp.float32)
    res = acc[:, 0:OUT]
    for l in range(1, K):
        res = jnp.maximum(res, acc[:, l * OUT:(l + 1) * OUT])
    out_ref[...] = res + b_ref[...]


def _conv(v, w2, brow):
    grid = (N // _BR3,)
    return pl.pallas_call(
        _conv_body,
        grid=grid,
        in_specs=[
            pl.BlockSpec((_BR3, K * D), lambda i: (i, 0)),
            pl.BlockSpec((K * D, K * OUT), lambda i: (0, 0)),
            pl.BlockSpec((1, OUT), lambda i: (0, 0)),
        ],
        out_specs=pl.BlockSpec((_BR3, OUT), lambda i: (i, 0)),
        out_shape=jax.ShapeDtypeStruct((N, OUT), jnp.float32),
    )(v, w2, brow)


def kernel(x, W, b):
    pcdT = x[0]
    pcd = pcdT.T

    idx16 = _topk_fast(pcdT, pcd)
    idx_fast = idx16[:, :K]
    overflow = jnp.max(idx16[:, K]) > 0
    idx = lax.cond(overflow,
                   lambda: _topk(pcdT, pcd),
                   lambda: idx_fast)

    g = _make_sc_gather()(pcd, idx.reshape(-1))
    v = g.reshape(N, K * D).astype(jnp.bfloat16)

    wm = W[:, :, 0].T
    eye = jnp.eye(K, dtype=jnp.float32)
    w2 = (wm[:, None, None, :] * eye[None, :, :, None]).reshape(D * K, K * OUT)

    return _conv(v, w2.astype(jnp.bfloat16), b.reshape(1, OUT))


# speedup vs baseline: 1.0871x; 1.0871x over previous
import functools

import jax
import jax.numpy as jnp
from jax import lax
from jax.experimental import pallas as pl
from jax.experimental.pallas import tpu as pltpu
from jax.experimental.pallas import tpu_sc as plsc

N = 8192
D = 64
K = 10
OUT = 128


_BR = 256


def _topk_body(pcdT_ref, pcd_blk_ref, idx_ref):
    pcdT = pcdT_ref[...]
    a = pcd_blk_ref[...]
    sq_col = jnp.sum(pcdT * pcdT, axis=0, keepdims=True)
    sq_row = jnp.sum(a * a, axis=1, keepdims=True)
    dot = jnp.dot(a, pcdT, preferred_element_type=jnp.float32)
    dd = (sq_row + sq_col) - 2.0 * dot

    iota_row = lax.broadcasted_iota(jnp.int32, (1, N), 1)
    big = jnp.int32(N)
    cols = []
    for _ in range(K):
        m = jnp.min(dd, axis=1, keepdims=True)
        cand = jnp.where(dd == m, iota_row, big)
        j = jnp.min(cand, axis=1, keepdims=True)
        cols.append(j)
        dd = jnp.where(iota_row == j, jnp.inf, dd)
    idx_ref[...] = jnp.concatenate(cols, axis=1)


def _topk(pcdT, pcd):
    grid = (N // _BR,)
    return pl.pallas_call(
        _topk_body,
        grid=grid,
        in_specs=[
            pl.BlockSpec((D, N), lambda i: (0, 0)),
            pl.BlockSpec((_BR, D), lambda i: (i, 0)),
        ],
        out_specs=pl.BlockSpec((_BR, K), lambda i: (i, 0)),
        out_shape=jax.ShapeDtypeStruct((N, K), jnp.int32),
    )(pcdT, pcd)


_DEPTH = 4
_G = N // 128


def _topk_fast_body(pcdT_ref, pcd_blk_ref, out_ref):
    pcdT = pcdT_ref[...]
    a = pcd_blk_ref[...]
    sq_col = jnp.sum(pcdT * pcdT, axis=0, keepdims=True)
    dot = jnp.dot(a, pcdT, preferred_element_type=jnp.float32)
    dd = sq_col - 2.0 * dot

    lane = lax.broadcasted_iota(jnp.int32, (1, 128), 1)
    inf = jnp.float32(jnp.inf)
    big = jnp.int32(1 << 30)

    sv = [jnp.full((_BR, 128), inf, jnp.float32) for _ in range(_DEPTH)]
    sa = [jnp.zeros((_BR, 128), jnp.int32) for _ in range(_DEPTH)]
    rej = jnp.full((_BR, 128), inf, jnp.float32)
    for g in range(_G):
        v = dd[:, g * 128:(g + 1) * 128]
        vid = jnp.full((1, 1), g, jnp.int32)
        for t in range(_DEPTH):
            c = v < sv[t]
            sv[t], v = jnp.where(c, v, sv[t]), jnp.where(c, sv[t], v)
            sa[t], vid = jnp.where(c, vid, sa[t]), jnp.where(c, sa[t], vid)
        rej = jnp.minimum(rej, v)

    gid = [sa[t] * 128 + lane for t in range(_DEPTH)]
    cv = jnp.concatenate(sv, axis=1)
    ci = jnp.concatenate(gid, axis=1)
    ct = cv.T
    it = ci.T

    rows = []
    m = None
    for _ in range(K):
        m = jnp.min(ct, axis=0, keepdims=True)
        cand = jnp.where(ct == m, it, big)
        j = jnp.min(cand, axis=0, keepdims=True)
        rows.append(j)
        ct = jnp.where(it == j, inf, ct)

    rejt = rej.T
    rejmin = jnp.min(rejt, axis=0, keepdims=True)
    rowflag = (rejmin <= m).astype(jnp.int32)
    for _ in range(16 - K):
        rows.append(rowflag)
    out_ref[...] = jnp.concatenate(rows, axis=0).T


def _topk_fast(pcdT, pcd):
    grid = (N // _BR,)
    return pl.pallas_call(
        _topk_fast_body,
        grid=grid,
        in_specs=[
            pl.BlockSpec((D, N), lambda i: (0, 0)),
            pl.BlockSpec((_BR, D), lambda i: (i, 0)),
        ],
        out_specs=pl.BlockSpec((_BR, 16), lambda i: (i, 0)),
        out_shape=jax.ShapeDtypeStruct((N, 16), jnp.int32),
    )(pcdT, pcd)


_B = N * K
_CHUNK = 1280


def _make_sc_gather():
    info = plsc.get_sparse_core_info()
    nc, ns = info.num_cores, info.num_subcores
    nw = nc * ns
    b_per_w = _B // nw
    n_chunks = b_per_w // _CHUNK
    mesh = plsc.VectorSubcoreMesh(core_axis_name="c", subcore_axis_name="s")

    @functools.partial(
        pl.kernel,
        mesh=mesh,
        compiler_params=pltpu.CompilerParams(use_tc_tiling_on_sc=False),
        out_type=jax.ShapeDtypeStruct((_B, D), jnp.float32),
        scratch_types=[
            pltpu.VMEM((_CHUNK,), jnp.int32),
            pltpu.VMEM((_CHUNK, D), jnp.float32),
            pltpu.SemaphoreType.DMA,
        ],
    )
    def gather(table_hbm, idx_hbm, out_hbm, idx_v, rows_v, sem):
        wid = lax.axis_index("s") * nc + lax.axis_index("c")
        base = wid * b_per_w
        for c in range(n_chunks):
            off = base + c * _CHUNK
            pltpu.sync_copy(idx_hbm.at[pl.ds(off, _CHUNK)], idx_v)
            pltpu.async_copy(table_hbm.at[idx_v], rows_v, sem).wait()
            pltpu.sync_copy(rows_v, out_hbm.at[pl.ds(off, _CHUNK)])

    return gather


_BR3 = 512


def _conv_body(v_ref, w2_ref, b_ref, out_ref):
    v = v_ref[...]
    w2 = w2_ref[...]
    acc = jnp.dot(v, w2, preferred_element_type=jnp.float32)
    res = acc[:, 0:OUT]
    for l in range(1, K):
        res = jnp.maximum(res, acc[:, l * OUT:(l + 1) * OUT])
    out_ref[...] = res + b_ref[...]


def _conv(v, w2, brow):
    grid = (N // _BR3,)
    return pl.pallas_call(
        _conv_body,
        grid=grid,
        in_specs=[
            pl.BlockSpec((_BR3, K * D), lambda i: (i, 0)),
            pl.BlockSpec((K * D, K * OUT), lambda i: (0, 0)),
            pl.BlockSpec((1, OUT), lambda i: (0, 0)),
        ],
        out_specs=pl.BlockSpec((_BR3, OUT), lambda i: (i, 0)),
        out_shape=jax.ShapeDtypeStruct((N, OUT), jnp.float32),
    )(v, w2, brow)


def kernel(x, W, b):
    pcdT = x[0]
    pcd = pcdT.T

    idx16 = _topk_fast(pcdT, pcd)
    idx_fast = idx16[:, :K]
    overflow = jnp.max(idx16[:, K]) > 0
    idx = lax.cond(overflow,
                   lambda: _topk(pcdT, pcd),
                   lambda: idx_fast)

    g = _make_sc_gather()(pcd, idx.reshape(-1))
    v = g.reshape(N, K * D)

    wm = W[:, :, 0].T
    eye = jnp.eye(K, dtype=jnp.float32)
    w2 = (wm[:, None, None, :] * eye[None, :, :, None]).reshape(D * K, K * OUT)

    return _conv(v, w2, b.reshape(1, OUT))


# speedup vs baseline: 1.1548x; 1.0623x over previous
import functools

import jax
import jax.numpy as jnp
from jax import lax
from jax.experimental import pallas as pl
from jax.experimental.pallas import tpu as pltpu
from jax.experimental.pallas import tpu_sc as plsc

N = 8192
D = 64
K = 10
OUT = 128


_BR = 256


def _topk_body(pcdT_ref, pcd_blk_ref, idx_ref):
    pcdT = pcdT_ref[...]
    a = pcd_blk_ref[...]
    sq_col = jnp.sum(pcdT * pcdT, axis=0, keepdims=True)
    sq_row = jnp.sum(a * a, axis=1, keepdims=True)
    dot = jnp.dot(a, pcdT, preferred_element_type=jnp.float32)
    dd = (sq_row + sq_col) - 2.0 * dot

    iota_row = lax.broadcasted_iota(jnp.int32, (1, N), 1)
    big = jnp.int32(N)
    cols = []
    for _ in range(K):
        m = jnp.min(dd, axis=1, keepdims=True)
        cand = jnp.where(dd == m, iota_row, big)
        j = jnp.min(cand, axis=1, keepdims=True)
        cols.append(j)
        dd = jnp.where(iota_row == j, jnp.inf, dd)
    idx_ref[...] = jnp.concatenate(cols, axis=1)


def _topk(pcdT, pcd):
    grid = (N // _BR,)
    return pl.pallas_call(
        _topk_body,
        grid=grid,
        in_specs=[
            pl.BlockSpec((D, N), lambda i: (0, 0)),
            pl.BlockSpec((_BR, D), lambda i: (i, 0)),
        ],
        out_specs=pl.BlockSpec((_BR, K), lambda i: (i, 0)),
        out_shape=jax.ShapeDtypeStruct((N, K), jnp.int32),
    )(pcdT, pcd)


_DEPTH = 4
_G = N // 128


def _topk_fast_body(pcdT_ref, pcd_blk_ref, out_ref):
    pcdT = pcdT_ref[...]
    a = pcd_blk_ref[...]
    sq_col = jnp.sum(pcdT * pcdT, axis=0, keepdims=True)
    dot = jnp.dot(a, pcdT, preferred_element_type=jnp.float32)
    dd = sq_col - 2.0 * dot

    lane = lax.broadcasted_iota(jnp.int32, (1, 128), 1)
    inf = jnp.float32(jnp.inf)
    big = jnp.int32(1 << 30)

    def _ce(av, ai, bv, bi):
        c = bv < av
        return (jnp.where(c, bv, av), jnp.where(c, bi, ai),
                jnp.where(c, av, bv), jnp.where(c, ai, bi))

    sv = [jnp.full((_BR, 128), inf, jnp.float32) for _ in range(_DEPTH)]
    sa = [jnp.zeros((_BR, 128), jnp.int32) for _ in range(_DEPTH)]
    rej = jnp.full((_BR, 128), inf, jnp.float32)
    for q in range(_G // 4):
        vs = [dd[:, (4 * q + r) * 128:(4 * q + r + 1) * 128] for r in range(4)]
        ids = [jnp.full((1, 1), 4 * q + r, jnp.int32) for r in range(4)]
        v0, v1, v2, v3 = vs
        i0, i1, i2, i3 = ids
        v0, i0, v1, i1 = _ce(v0, i0, v1, i1)
        v2, i2, v3, i3 = _ce(v2, i2, v3, i3)
        v0, i0, v2, i2 = _ce(v0, i0, v2, i2)
        v1, i1, v3, i3 = _ce(v1, i1, v3, i3)
        v1, i1, v2, i2 = _ce(v1, i1, v2, i2)
        m1, mi1, x1, _ = _ce(sv[0], sa[0], v3, i3)
        m2, mi2, x2, _ = _ce(sv[1], sa[1], v2, i2)
        m3, mi3, x3, _ = _ce(sv[2], sa[2], v1, i1)
        m4, mi4, x4, _ = _ce(sv[3], sa[3], v0, i0)
        rej = jnp.minimum(rej, jnp.minimum(jnp.minimum(x1, x2),
                                           jnp.minimum(x3, x4)))
        m1, mi1, m3, mi3 = _ce(m1, mi1, m3, mi3)
        m2, mi2, m4, mi4 = _ce(m2, mi2, m4, mi4)
        m1, mi1, m2, mi2 = _ce(m1, mi1, m2, mi2)
        m3, mi3, m4, mi4 = _ce(m3, mi3, m4, mi4)
        sv = [m1, m2, m3, m4]
        sa = [mi1, mi2, mi3, mi4]

    gid = [sa[t] * 128 + lane for t in range(_DEPTH)]
    cv = jnp.concatenate(sv, axis=1)
    ci = jnp.concatenate(gid, axis=1)
    ct = cv.T
    it = ci.T

    rows = []
    m = None
    for _ in range(K):
        m = jnp.min(ct, axis=0, keepdims=True)
        cand = jnp.where(ct == m, it, big)
        j = jnp.min(cand, axis=0, keepdims=True)
        rows.append(j)
        ct = jnp.where(it == j, inf, ct)

    rejt = rej.T
    rejmin = jnp.min(rejt, axis=0, keepdims=True)
    rowflag = (rejmin <= m).astype(jnp.int32)
    for _ in range(16 - K):
        rows.append(rowflag)
    out_ref[...] = jnp.concatenate(rows, axis=0).T


def _topk_fast(pcdT, pcd):
    grid = (N // _BR,)
    return pl.pallas_call(
        _topk_fast_body,
        grid=grid,
        in_specs=[
            pl.BlockSpec((D, N), lambda i: (0, 0)),
            pl.BlockSpec((_BR, D), lambda i: (i, 0)),
        ],
        out_specs=pl.BlockSpec((_BR, 16), lambda i: (i, 0)),
        out_shape=jax.ShapeDtypeStruct((N, 16), jnp.int32),
    )(pcdT, pcd)


_B = N * K
_CHUNK = 1280


def _make_sc_gather():
    info = plsc.get_sparse_core_info()
    nc, ns = info.num_cores, info.num_subcores
    nw = nc * ns
    b_per_w = _B // nw
    n_chunks = b_per_w // _CHUNK
    mesh = plsc.VectorSubcoreMesh(core_axis_name="c", subcore_axis_name="s")

    @functools.partial(
        pl.kernel,
        mesh=mesh,
        compiler_params=pltpu.CompilerParams(use_tc_tiling_on_sc=False),
        out_type=jax.ShapeDtypeStruct((_B, D), jnp.float32),
        scratch_types=[
            pltpu.VMEM((_CHUNK,), jnp.int32),
            pltpu.VMEM((_CHUNK, D), jnp.float32),
            pltpu.SemaphoreType.DMA,
        ],
    )
    def gather(table_hbm, idx_hbm, out_hbm, idx_v, rows_v, sem):
        wid = lax.axis_index("s") * nc + lax.axis_index("c")
        base = wid * b_per_w
        for c in range(n_chunks):
            off = base + c * _CHUNK
            pltpu.sync_copy(idx_hbm.at[pl.ds(off, _CHUNK)], idx_v)
            pltpu.async_copy(table_hbm.at[idx_v], rows_v, sem).wait()
            pltpu.sync_copy(rows_v, out_hbm.at[pl.ds(off, _CHUNK)])

    return gather


_BR3 = 512


def _conv_body(v_ref, w2_ref, b_ref, out_ref):
    v = v_ref[...]
    w2 = w2_ref[...]
    acc = jnp.dot(v, w2, preferred_element_type=jnp.float32)
    res = acc[:, 0:OUT]
    for l in range(1, K):
        res = jnp.maximum(res, acc[:, l * OUT:(l + 1) * OUT])
    out_ref[...] = res + b_ref[...]


def _conv(v, w2, brow):
    grid = (N // _BR3,)
    return pl.pallas_call(
        _conv_body,
        grid=grid,
        in_specs=[
            pl.BlockSpec((_BR3, K * D), lambda i: (i, 0)),
            pl.BlockSpec((K * D, K * OUT), lambda i: (0, 0)),
            pl.BlockSpec((1, OUT), lambda i: (0, 0)),
        ],
        out_specs=pl.BlockSpec((_BR3, OUT), lambda i: (i, 0)),
        out_shape=jax.ShapeDtypeStruct((N, OUT), jnp.float32),
    )(v, w2, brow)


def kernel(x, W, b):
    pcdT = x[0]
    pcd = pcdT.T

    idx16 = _topk_fast(pcdT, pcd)
    idx_fast = idx16[:, :K]
    overflow = jnp.max(idx16[:, K]) > 0
    idx = lax.cond(overflow,
                   lambda: _topk(pcdT, pcd),
                   lambda: idx_fast)

    g = _make_sc_gather()(pcd, idx.reshape(-1))
    v = g.reshape(N, K * D)

    wm = W[:, :, 0].T
    eye = jnp.eye(K, dtype=jnp.float32)
    w2 = (wm[:, None, None, :] * eye[None, :, :, None]).reshape(D * K, K * OUT)

    return _conv(v, w2, b.reshape(1, OUT))


# speedup vs baseline: 1.2002x; 1.0392x over previous
import functools

import jax
import jax.numpy as jnp
from jax import lax
from jax.experimental import pallas as pl
from jax.experimental.pallas import tpu as pltpu
from jax.experimental.pallas import tpu_sc as plsc

N = 8192
D = 64
K = 10
OUT = 128


_BR = 256


def _topk_body(pcdT_ref, pcd_blk_ref, idx_ref):
    pcdT = pcdT_ref[...]
    a = pcd_blk_ref[...]
    sq_col = jnp.sum(pcdT * pcdT, axis=0, keepdims=True)
    sq_row = jnp.sum(a * a, axis=1, keepdims=True)
    dot = jnp.dot(a, pcdT, preferred_element_type=jnp.float32)
    dd = (sq_row + sq_col) - 2.0 * dot

    iota_row = lax.broadcasted_iota(jnp.int32, (1, N), 1)
    big = jnp.int32(N)
    cols = []
    for _ in range(K):
        m = jnp.min(dd, axis=1, keepdims=True)
        cand = jnp.where(dd == m, iota_row, big)
        j = jnp.min(cand, axis=1, keepdims=True)
        cols.append(j)
        dd = jnp.where(iota_row == j, jnp.inf, dd)
    idx_ref[...] = jnp.concatenate(cols, axis=1)


def _topk(pcdT, pcd):
    grid = (N // _BR,)
    return pl.pallas_call(
        _topk_body,
        grid=grid,
        in_specs=[
            pl.BlockSpec((D, N), lambda i: (0, 0)),
            pl.BlockSpec((_BR, D), lambda i: (i, 0)),
        ],
        out_specs=pl.BlockSpec((_BR, K), lambda i: (i, 0)),
        out_shape=jax.ShapeDtypeStruct((N, K), jnp.int32),
    )(pcdT, pcd)


_DEPTH = 4
_G = N // 128


def _topk_fast_body(pcdT_ref, pcd_blk_ref, out_ref):
    pcdT = pcdT_ref[...]
    a = pcd_blk_ref[...]
    sq_col = jnp.sum(pcdT * pcdT, axis=0, keepdims=True)
    aa = jnp.concatenate([a * -2.0, jnp.ones((_BR, 1), jnp.float32)], axis=1)
    pp = jnp.concatenate([pcdT, sq_col], axis=0)
    dd = jnp.dot(aa, pp, preferred_element_type=jnp.float32)

    lane = lax.broadcasted_iota(jnp.int32, (1, 128), 1)
    inf = jnp.float32(jnp.inf)
    big = jnp.int32(1 << 30)

    def _ce(av, ai, bv, bi):
        c = bv < av
        return (jnp.where(c, bv, av), jnp.where(c, bi, ai),
                jnp.where(c, av, bv), jnp.where(c, ai, bi))

    sv = [jnp.full((_BR, 128), inf, jnp.float32) for _ in range(_DEPTH)]
    sa = [jnp.zeros((_BR, 128), jnp.int32) for _ in range(_DEPTH)]
    rej = jnp.full((_BR, 128), inf, jnp.float32)
    for q in range(_G // 4):
        vs = [dd[:, (4 * q + r) * 128:(4 * q + r + 1) * 128] for r in range(4)]
        ids = [jnp.full((1, 1), 4 * q + r, jnp.int32) for r in range(4)]
        v0, v1, v2, v3 = vs
        i0, i1, i2, i3 = ids
        v0, i0, v1, i1 = _ce(v0, i0, v1, i1)
        v2, i2, v3, i3 = _ce(v2, i2, v3, i3)
        v0, i0, v2, i2 = _ce(v0, i0, v2, i2)
        v1, i1, v3, i3 = _ce(v1, i1, v3, i3)
        v1, i1, v2, i2 = _ce(v1, i1, v2, i2)
        m1, mi1, x1, _ = _ce(sv[0], sa[0], v3, i3)
        m2, mi2, x2, _ = _ce(sv[1], sa[1], v2, i2)
        m3, mi3, x3, _ = _ce(sv[2], sa[2], v1, i1)
        m4, mi4, x4, _ = _ce(sv[3], sa[3], v0, i0)
        rej = jnp.minimum(rej, jnp.minimum(jnp.minimum(x1, x2),
                                           jnp.minimum(x3, x4)))
        m1, mi1, m3, mi3 = _ce(m1, mi1, m3, mi3)
        m2, mi2, m4, mi4 = _ce(m2, mi2, m4, mi4)
        m1, mi1, m2, mi2 = _ce(m1, mi1, m2, mi2)
        m3, mi3, m4, mi4 = _ce(m3, mi3, m4, mi4)
        sv = [m1, m2, m3, m4]
        sa = [mi1, mi2, mi3, mi4]

    gid = [sa[t] * 128 + lane for t in range(_DEPTH)]
    cv = jnp.concatenate(sv, axis=1)
    ci = jnp.concatenate(gid, axis=1)
    ct = cv.T
    it = ci.T

    rows = []
    m = None
    for _ in range(K):
        m = jnp.min(ct, axis=0, keepdims=True)
        cand = jnp.where(ct == m, it, big)
        j = jnp.min(cand, axis=0, keepdims=True)
        rows.append(j)
        ct = jnp.where(it == j, inf, ct)

    rejt = rej.T
    rejmin = jnp.min(rejt, axis=0, keepdims=True)
    rowflag = (rejmin <= m).astype(jnp.int32)
    for _ in range(16 - K):
        rows.append(rowflag)
    out_ref[...] = jnp.concatenate(rows, axis=0).T


def _topk_fast(pcdT, pcd):
    grid = (N // _BR,)
    return pl.pallas_call(
        _topk_fast_body,
        grid=grid,
        in_specs=[
            pl.BlockSpec((D, N), lambda i: (0, 0)),
            pl.BlockSpec((_BR, D), lambda i: (i, 0)),
        ],
        out_specs=pl.BlockSpec((_BR, 16), lambda i: (i, 0)),
        out_shape=jax.ShapeDtypeStruct((N, 16), jnp.int32),
    )(pcdT, pcd)


_B = N * K
_CHUNK = 1280


def _make_sc_gather():
    info = plsc.get_sparse_core_info()
    nc, ns = info.num_cores, info.num_subcores
    nw = nc * ns
    b_per_w = _B // nw
    n_chunks = b_per_w // _CHUNK
    mesh = plsc.VectorSubcoreMesh(core_axis_name="c", subcore_axis_name="s")

    @functools.partial(
        pl.kernel,
        mesh=mesh,
        compiler_params=pltpu.CompilerParams(use_tc_tiling_on_sc=False),
        out_type=jax.ShapeDtypeStruct((_B, D), jnp.float32),
        scratch_types=[
            pltpu.VMEM((_CHUNK,), jnp.int32),
            pltpu.VMEM((_CHUNK, D), jnp.float32),
            pltpu.SemaphoreType.DMA,
        ],
    )
    def gather(table_hbm, idx_hbm, out_hbm, idx_v, rows_v, sem):
        wid = lax.axis_index("s") * nc + lax.axis_index("c")
        base = wid * b_per_w
        for c in range(n_chunks):
            off = base + c * _CHUNK
            pltpu.sync_copy(idx_hbm.at[pl.ds(off, _CHUNK)], idx_v)
            pltpu.async_copy(table_hbm.at[idx_v], rows_v, sem).wait()
            pltpu.sync_copy(rows_v, out_hbm.at[pl.ds(off, _CHUNK)])

    return gather


_BR3 = 512


def _conv_body(v_ref, w2_ref, b_ref, out_ref):
    v = v_ref[...]
    w2 = w2_ref[...]
    acc = jnp.dot(v, w2, preferred_element_type=jnp.float32)
    res = acc[:, 0:OUT]
    for l in range(1, K):
        res = jnp.maximum(res, acc[:, l * OUT:(l + 1) * OUT])
    out_ref[...] = res + b_ref[...]


def _conv(v, w2, brow):
    grid = (N // _BR3,)
    return pl.pallas_call(
        _conv_body,
        grid=grid,
        in_specs=[
            pl.BlockSpec((_BR3, K * D), lambda i: (i, 0)),
            pl.BlockSpec((K * D, K * OUT), lambda i: (0, 0)),
            pl.BlockSpec((1, OUT), lambda i: (0, 0)),
        ],
        out_specs=pl.BlockSpec((_BR3, OUT), lambda i: (i, 0)),
        out_shape=jax.ShapeDtypeStruct((N, OUT), jnp.float32),
    )(v, w2, brow)


def kernel(x, W, b):
    pcdT = x[0]
    pcd = pcdT.T

    idx16 = _topk_fast(pcdT, pcd)
    idx_fast = idx16[:, :K]
    overflow = jnp.max(idx16[:, K]) > 0
    idx = lax.cond(overflow,
                   lambda: _topk(pcdT, pcd),
                   lambda: idx_fast)

    g = _make_sc_gather()(pcd, idx.reshape(-1))
    v = g.reshape(N, K * D)

    wm = W[:, :, 0].T
    eye = jnp.eye(K, dtype=jnp.float32)
    w2 = (wm[:, None, None, :] * eye[None, :, :, None]).reshape(D * K, K * OUT)

    return _conv(v, w2, b.reshape(1, OUT))
